# Initial kernel scaffold; baseline (speedup 1.0000x reference)
import jax
import jax.numpy as jnp
from jax.experimental import pallas as pl


def kernel(question, img_nodes, img_edge_index, img_edge_feat, kg_nodes, kg_edge_index, kg_edge_feat, params):
    raise NotImplementedError("write your pallas kernel here")


# speedup vs baseline: 15.1499x; 15.1499x over previous
import functools

import jax
import jax.numpy as jnp
import numpy as np
from jax.experimental import pallas as pl
from jax.experimental.pallas import tpu as pltpu

H = 8
EB = 512


def _cs_body(iei_ref, ief_ref, kei_ref, kef_ref, ci_ref, si_ref, ck_ref, sk_ref):
    n = ci_ref.shape[1]
    e = iei_ref.shape[2]
    de = ief_ref.shape[2]
    eb = min(EB, e)
    iota_col = jax.lax.broadcasted_iota(jnp.int32, (n, eb), 0)
    for ei_ref, ef_ref, c_ref, s_ref in (
        (iei_ref, ief_ref, ci_ref, si_ref),
        (kei_ref, kef_ref, ck_ref, sk_ref),
    ):
        c_acc = jnp.zeros((n, n), jnp.float32)
        s_acc = jnp.zeros((n, de), jnp.float32)
        for b in range(e // eb):
            src = ei_ref[0, 0:1, pl.ds(b * eb, eb)]
            dst = ei_ref[0, 1:2, pl.ds(b * eb, eb)]
            a_dst = (iota_col == dst).astype(jnp.bfloat16)
            a_srct = (iota_col == src).astype(jnp.bfloat16)
            c_acc = c_acc + jax.lax.dot_general(
                a_dst, a_srct, (((1,), (1,)), ((), ())),
                preferred_element_type=jnp.float32)
            efb = ef_ref[0, pl.ds(b * eb, eb), :]
            s_acc = s_acc + jnp.dot(a_dst.astype(jnp.float32), efb,
                                    preferred_element_type=jnp.float32)
        c_ref[0] = c_acc
        s_ref[0] = s_acc


def _build_cs_sized(img_ei, img_ef, kg_ei, kg_ef, n):
    bsz, _, e = img_ei.shape
    de = img_ef.shape[2]
    grid = (bsz,)
    ei_spec = pl.BlockSpec((1, 2, e), lambda b: (b, 0, 0))
    ef_spec = pl.BlockSpec((1, e, de), lambda b: (b, 0, 0))
    c_spec = pl.BlockSpec((1, n, n), lambda b: (b, 0, 0))
    s_spec = pl.BlockSpec((1, n, de), lambda b: (b, 0, 0))
    out_shape = (
        jax.ShapeDtypeStruct((bsz, n, n), jnp.float32),
        jax.ShapeDtypeStruct((bsz, n, de), jnp.float32),
        jax.ShapeDtypeStruct((bsz, n, n), jnp.float32),
        jax.ShapeDtypeStruct((bsz, n, de), jnp.float32),
    )
    return pl.pallas_call(
        _cs_body,
        grid=grid,
        in_specs=[ei_spec, ef_spec, ei_spec, ef_spec],
        out_specs=(c_spec, s_spec, c_spec, s_spec),
        out_shape=out_shape,
        compiler_params=pltpu.CompilerParams(
            dimension_semantics=("parallel",)),
    )(img_ei, img_ef, kg_ei, kg_ef)


def _ln(x, w, b):
    mu = jnp.mean(x, axis=-1, keepdims=True)
    xc = x - mu
    var = jnp.mean(xc * xc, axis=-1, keepdims=True)
    return xc * jax.lax.rsqrt(var + 1e-5) * w + b


def _mha_ffn(q, kv, w):
    (wq, bq, wk, bk, wv, bv, wo, bo, ln1w, ln1b,
     f1w, f1b, f2w, f2b, ln2w, ln2b) = w
    d = q.shape[1]
    dh = d // H
    qh = jnp.dot(q, wq, preferred_element_type=jnp.float32) + bq
    kh = jnp.dot(kv, wk, preferred_element_type=jnp.float32) + bk
    vh = jnp.dot(kv, wv, preferred_element_type=jnp.float32) + bv
    scale = 1.0 / np.sqrt(dh)
    outs = []
    for h in range(H):
        qs = qh[:, h * dh:(h + 1) * dh]
        ks = kh[:, h * dh:(h + 1) * dh]
        vs = vh[:, h * dh:(h + 1) * dh]
        s = jax.lax.dot_general(qs, ks, (((1,), (1,)), ((), ())),
                                preferred_element_type=jnp.float32) * scale
        m = jnp.max(s, axis=-1, keepdims=True)
        p = jnp.exp(s - m)
        p = p / jnp.sum(p, axis=-1, keepdims=True)
        outs.append(jnp.dot(p, vs, preferred_element_type=jnp.float32))
    att = jnp.concatenate(outs, axis=1)
    o = jnp.dot(att, wo, preferred_element_type=jnp.float32) + bo
    x = _ln(q + o, ln1w, ln1b)
    mid = jax.nn.relu(jnp.dot(x, f1w, preferred_element_type=jnp.float32) + f1b)
    ffn = jnp.dot(mid, f2w, preferred_element_type=jnp.float32) + f2b
    return _ln(x + ffn, ln2w, ln2b)


def _gcn_f(nodes, c, s, qrow, w):
    wn, bn, wm, bm, wr, br, qpw, qpb = w
    m = jnp.dot(nodes, wm, preferred_element_type=jnp.float32) + bm
    deg = jnp.sum(c, axis=1, keepdims=True)
    agg = (jnp.dot(c, m, preferred_element_type=jnp.float32)
           + jnp.dot(s, wr, preferred_element_type=jnp.float32)
           + deg * br)
    h = jax.nn.relu(jnp.dot(nodes, wn, preferred_element_type=jnp.float32)
                    + bn + agg)
    qp = jnp.dot(qrow, qpw, preferred_element_type=jnp.float32) + qpb
    return jnp.tanh(qp + h)


def _main_body(nw, *refs):
    q_ref, ni_ref, ci_ref, si_ref, nk_ref, ck_ref, sk_ref = refs[:7]
    wrefs = refs[7:7 + nw]
    oi_ref, ok_ref = refs[7 + nw:]
    w = [r[...] for r in wrefs]
    gi, gk = w[:8], w[8:16]
    cai, cak = w[16:32], w[32:48]
    qrow = q_ref[0]
    nodes_i = ni_ref[0]
    nodes_k = nk_ref[0]
    f_i = _gcn_f(nodes_i, ci_ref[0], si_ref[0], qrow, gi)
    f_k = _gcn_f(nodes_k, ck_ref[0], sk_ref[0], qrow, gk)
    o_i = _mha_ffn(f_i, f_k, cai)
    o_k = _mha_ffn(f_k, f_i, cak)
    r_i = nodes_i + o_i
    r_k = nodes_k + o_k
    ninv_i = jnp.maximum(jnp.sqrt(jnp.sum(r_i * r_i, axis=-1, keepdims=True)),
                         1e-12)
    ninv_k = jnp.maximum(jnp.sqrt(jnp.sum(r_k * r_k, axis=-1, keepdims=True)),
                         1e-12)
    oi_ref[0] = r_i / ninv_i
    ok_ref[0] = r_k / ninv_k


def _full(shape_arr):
    ndim = shape_arr.ndim
    return pl.BlockSpec(shape_arr.shape, lambda b: (0,) * ndim)


def kernel(question, img_nodes, img_edge_index, img_edge_feat,
           kg_nodes, kg_edge_index, kg_edge_feat, params):
    bsz, nq = question.shape
    n, d = img_nodes.shape[1:]
    c_i, s_i, c_k, s_k = _build_cs_sized(
        img_edge_index, img_edge_feat, kg_edge_index, kg_edge_feat, n)

    def row(v):
        return v.reshape(1, -1)

    def gcn_w(p, qp):
        g = params[p]
        return [g["Wn"][0], row(g["Wn"][1]), g["Wm"][0], row(g["Wm"][1]),
                g["Wr"][0], row(g["Wr"][1]), params[qp][0], row(params[qp][1])]

    def ca_w(p):
        c = params[p]
        return [c["Wq"][0], row(c["Wq"][1]), c["Wk"][0], row(c["Wk"][1]),
                c["Wv"][0], row(c["Wv"][1]), c["Wo"][0], row(c["Wo"][1]),
                row(c["ln1"][0]), row(c["ln1"][1]),
                c["f1"][0], row(c["f1"][1]), c["f2"][0], row(c["f2"][1]),
                row(c["ln2"][0]), row(c["ln2"][1])]

    weights = (gcn_w("gcn_img", "q_img_prj") + gcn_w("gcn_kg", "q_kg_prj")
               + ca_w("ca_img") + ca_w("ca_kg"))
    nw = len(weights)
    de = img_edge_feat.shape[2]

    in_specs = [
        pl.BlockSpec((1, 1, nq), lambda b: (b, 0, 0)),
        pl.BlockSpec((1, n, d), lambda b: (b, 0, 0)),
        pl.BlockSpec((1, n, n), lambda b: (b, 0, 0)),
        pl.BlockSpec((1, n, de), lambda b: (b, 0, 0)),
        pl.BlockSpec((1, n, d), lambda b: (b, 0, 0)),
        pl.BlockSpec((1, n, n), lambda b: (b, 0, 0)),
        pl.BlockSpec((1, n, de), lambda b: (b, 0, 0)),
    ] + [_full(wa) for wa in weights]
    out_spec = pl.BlockSpec((1, n, d), lambda b: (b, 0, 0))
    out_shape = (jax.ShapeDtypeStruct((bsz, n, d), jnp.float32),
                 jax.ShapeDtypeStruct((bsz, n, d), jnp.float32))
    o_i, o_k = pl.pallas_call(
        functools.partial(_main_body, nw),
        grid=(bsz,),
        in_specs=in_specs,
        out_specs=(out_spec, out_spec),
        out_shape=out_shape,
        compiler_params=pltpu.CompilerParams(
            dimension_semantics=("parallel",)),
    )(question.reshape(bsz, 1, nq), img_nodes, c_i, s_i,
      kg_nodes, c_k, s_k, *weights)
    return o_i, o_k
